# baseline (speedup 1.0000x reference)
import functools

import jax
import jax.numpy as jnp
from jax import lax
from jax.experimental import pallas as pl
from jax.experimental.pallas import tpu as pltpu
from jax.experimental.pallas import tpu_sc as plsc

D = 64
SEQ_LEN = 200
CHUNK = 512
GROUP = 128
GROUPS_PER_CHUNK = CHUNK // GROUP
POS_REP = 4
N_CORES = 2
N_SUBCORES = 16
N_WORKERS = N_CORES * N_SUBCORES


@functools.lru_cache(maxsize=None)
def _build(total_rows):
    rows_per_w = total_rows // N_WORKERS
    chunks_per_w = rows_per_w // CHUNK
    n_pairs = chunks_per_w // 2
    mesh = plsc.VectorSubcoreMesh(core_axis_name="c", subcore_axis_name="s")

    @functools.partial(
        pl.kernel,
        mesh=mesh,
        out_type=jax.ShapeDtypeStruct((total_rows, D), jnp.float32),
        compiler_params=pltpu.CompilerParams(use_tc_tiling_on_sc=False),
        scratch_types=[
            pltpu.VMEM((2 * GROUPS_PER_CHUNK, GROUP), jnp.int32),
            pltpu.VMEM((CHUNK, D), jnp.float32),
            pltpu.VMEM((CHUNK, D), jnp.float32),
            pltpu.VMEM((POS_REP * SEQ_LEN, D), jnp.float32),
            pltpu.SemaphoreType.DMA,
            pltpu.SemaphoreType.DMA,
            pltpu.SemaphoreType.DMA,
            pltpu.SemaphoreType.DMA,
        ],
    )
    def emb(x_hbm, tok_hbm, pos_hbm, out_hbm,
            idx_buf, row_a, row_b, posbuf,
            gsem_a, gsem_b, osem_a, osem_b):
        wid = lax.axis_index("s") * N_CORES + lax.axis_index("c")
        base_row = wid * rows_per_w

        for k in range(POS_REP):
            pltpu.sync_copy(pos_hbm, posbuf.at[pl.ds(k * SEQ_LEN, SEQ_LEN)])

        def stage_idx_pair(i):
            grp0 = pl.multiple_of((base_row + 2 * i * CHUNK) // GROUP, 8)
            pltpu.sync_copy(x_hbm.at[pl.ds(grp0, 2 * GROUPS_PER_CHUNK)], idx_buf)

        def start_gathers(half, row_buf, gsem):
            for j in range(GROUPS_PER_CHUNK):
                pltpu.async_copy(
                    tok_hbm.at[idx_buf.at[half * GROUPS_PER_CHUNK + j]],
                    row_buf.at[pl.ds(j * GROUP, GROUP)],
                    gsem,
                )

        def wait_gather(row_buf, gsem):
            pltpu.make_async_copy(tok_hbm.at[pl.ds(0, CHUNK)], row_buf, gsem).wait()

        def add_pos(g, row_buf):
            t0 = lax.rem(g * CHUNK, SEQ_LEN)

            def rbody(r, carry):
                pr = t0 + r
                for c in range(D // 16):
                    sl = pl.ds(c * 16, 16)
                    plsc.addupdate(row_buf.at[r, sl], posbuf[pr, sl])
                return carry

            lax.fori_loop(0, CHUNK, rbody, 0, unroll=2)

        def start_out(g, row_buf, osem):
            row0 = pl.multiple_of(base_row + g * CHUNK, 8)
            pltpu.async_copy(row_buf, out_hbm.at[pl.ds(row0, CHUNK)], osem)

        def wait_out(g, row_buf, osem):
            row0 = pl.multiple_of(base_row + g * CHUNK, 8)
            pltpu.make_async_copy(row_buf, out_hbm.at[pl.ds(row0, CHUNK)], osem).wait()

        def pair(i, carry):
            ga = 2 * i
            gb = 2 * i + 1

            stage_idx_pair(i)

            @pl.when(i > 0)
            def _():
                wait_out(ga - 2, row_a, osem_a)

            start_gathers(0, row_a, gsem_a)

            @pl.when(i > 0)
            def _():
                wait_out(gb - 2, row_b, osem_b)

            start_gathers(1, row_b, gsem_b)

            wait_gather(row_a, gsem_a)
            add_pos(ga, row_a)
            start_out(ga, row_a, osem_a)

            wait_gather(row_b, gsem_b)
            add_pos(gb, row_b)
            start_out(gb, row_b, osem_b)
            return carry

        lax.fori_loop(0, n_pairs, pair, 0)
        wait_out(2 * n_pairs - 2, row_a, osem_a)
        wait_out(2 * n_pairs - 1, row_b, osem_b)

    return emb


def kernel(x, token_table, pos_table):
    b, t = x.shape
    total_rows = b * t
    x_flat = x.astype(jnp.int32).reshape(total_rows // GROUP, GROUP)
    out = _build(total_rows)(x_flat, token_table, pos_table)
    return out.reshape(b, t, D)


# speedup vs baseline: 1.5042x; 1.5042x over previous
import functools

import jax
import jax.numpy as jnp
from jax import lax
from jax.experimental import pallas as pl
from jax.experimental.pallas import tpu as pltpu
from jax.experimental.pallas import tpu_sc as plsc

D = 64
PAD_D = 128
SEQ_LEN = 200
BLK = 128
TG = 8
N_CORES = 2
N_SUBCORES = 16
N_WORKERS = N_CORES * N_SUBCORES
TSTRIDE = 129


@functools.lru_cache(maxsize=None)
def _build(batch, vocab):
    n_tg = SEQ_LEN // TG
    nb = batch // BLK
    mesh = plsc.VectorSubcoreMesh(core_axis_name="c", subcore_axis_name="s")

    @functools.partial(
        pl.kernel,
        mesh=mesh,
        out_type=jax.ShapeDtypeStruct((SEQ_LEN, D // 8, nb, 8, BLK), jnp.float32),
        compiler_params=pltpu.CompilerParams(
            use_tc_tiling_on_sc=False, needs_layout_passes=False
        ),
        scratch_types=[
            pltpu.VMEM((TG, BLK), jnp.int32),
            pltpu.VMEM((TG, BLK), jnp.int32),
            pltpu.VMEM((BLK, D), jnp.float32),
            pltpu.VMEM((BLK, D), jnp.float32),
            pltpu.VMEM((D, TSTRIDE), jnp.float32),
            pltpu.VMEM((D, TSTRIDE), jnp.float32),
            pltpu.VMEM((SEQ_LEN, PAD_D), jnp.float32),
            pltpu.SemaphoreType.DMA,
            pltpu.SemaphoreType.DMA,
            pltpu.SemaphoreType.DMA,
            pltpu.SemaphoreType.DMA,
            pltpu.SemaphoreType.DMA,
        ],
    )
    def emb(xT_hbm, tbl_hbm, pos_hbm, outT_hbm,
            idx_a, idx_b, rows_a, rows_b, trans_a, trans_b, posbuf,
            isem, gsem_a, gsem_b, osem_a, osem_b):
        wid = lax.axis_index("s") * N_CORES + lax.axis_index("c")
        b0 = pl.multiple_of(wid * BLK, BLK)
        idx_bufs = (idx_a, idx_b)
        rows_bufs = (rows_a, rows_b)
        trans_bufs = (trans_a, trans_b)
        gsems = (gsem_a, gsem_b)
        osems = (osem_a, osem_b)

        pltpu.sync_copy(pos_hbm, posbuf)

        iota = lax.iota(jnp.int32, 16)
        trow = [iota + 16 * c for c in range(D // 16)]

        def stage_idx(tg, buf):
            t0 = pl.multiple_of(tg * TG, TG)
            return pltpu.make_async_copy(
                xT_hbm.at[pl.ds(t0, TG), pl.ds(b0, BLK)], buf, isem
            )

        def scale_idx(buf):
            for r in range(TG):
                for c in range(BLK // 16):
                    sl = pl.ds(16 * c, 16)
                    buf[r, sl] = buf[r, sl] * 2

        def start_gather(idx_buf, k, rows_buf, gsem):
            pltpu.async_copy(tbl_hbm.at[idx_buf.at[k]], rows_buf, gsem)

        def wait_gather(rows_buf, gsem):
            pltpu.make_async_copy(
                tbl_hbm.at[pl.ds(0, BLK)], rows_buf, gsem
            ).wait()

        def out_starts(t, trans_buf, osem):
            for i in range(D // 8):
                pltpu.async_copy(
                    trans_buf.at[pl.ds(8 * i, 8), pl.ds(0, BLK)],
                    outT_hbm.at[t, i, wid],
                    osem,
                )

        def out_wait(t, trans_buf, osem):
            for i in range(D // 8):
                pltpu.make_async_copy(
                    trans_buf.at[pl.ds(8 * i, 8), pl.ds(0, BLK)],
                    outT_hbm.at[t, i, wid],
                    osem,
                ).wait()

        def compute(t, rows_buf, trans_buf):
            posv = [posbuf[t, pl.ds(16 * c, 16)] for c in range(D // 16)]

            def tok_body(tok, carry):
                col = jnp.full((16,), tok, jnp.int32)
                for c in range(D // 16):
                    v = rows_buf[tok, pl.ds(16 * c, 16)] + posv[c]
                    plsc.store_scatter(trans_buf, [trow[c], col], v)
                return carry

            lax.fori_loop(0, BLK, tok_body, 0)

        stage_idx(0, idx_a).start()
        stage_idx(0, idx_a).wait()
        scale_idx(idx_a)
        start_gather(idx_a, 0, rows_a, gsem_a)

        def tg_body(tg, carry):
            def one_tg(cur, nxt):
                @pl.when(tg + 1 < n_tg)
                def _():
                    stage_idx(tg + 1, nxt).start()

                for k in range(TG):
                    t = tg * TG + k
                    p = k & 1
                    q = 1 - p
                    wait_gather(rows_bufs[p], gsems[p])
                    if k < TG - 1:
                        start_gather(cur, k + 1, rows_bufs[q], gsems[q])
                    else:
                        @pl.when(tg + 1 < n_tg)
                        def _():
                            stage_idx(tg + 1, nxt).wait()
                            scale_idx(nxt)
                            start_gather(nxt, 0, rows_bufs[q], gsems[q])

                    @pl.when(t >= 2)
                    def _():
                        out_wait(t - 2, trans_bufs[p], osems[p])

                    compute(t, rows_bufs[p], trans_bufs[p])
                    out_starts(t, trans_bufs[p], osems[p])

            @pl.when(lax.rem(tg, 2) == 0)
            def _():
                one_tg(idx_a, idx_b)

            @pl.when(lax.rem(tg, 2) == 1)
            def _():
                one_tg(idx_b, idx_a)

            return carry

        lax.fori_loop(0, n_tg, tg_body, 0)
        out_wait(SEQ_LEN - 2, trans_a, osem_a)
        out_wait(SEQ_LEN - 1, trans_b, osem_b)

    return emb


def kernel(x, token_table, pos_table):
    b, t = x.shape
    vocab = token_table.shape[0]
    xT = x.astype(jnp.int32).T
    tbl = jnp.pad(token_table, ((0, 0), (0, PAD_D - D))).reshape(2 * vocab, D)
    pos = jnp.pad(pos_table, ((0, 0), (0, PAD_D - D)))
    out5 = _build(b, vocab)(xT, tbl, pos)
    return out5.transpose(2, 4, 0, 1, 3).reshape(b, t, D)
